# Initial kernel scaffold; baseline (speedup 1.0000x reference)
import jax
import jax.numpy as jnp
from jax.experimental import pallas as pl


def kernel(grid_nfeat, mesh_nfeat, edge_index, grid2mesh_efeat, params):
    raise NotImplementedError("write your pallas kernel here")


# speedup vs baseline: 3.0420x; 3.0420x over previous
import functools

import jax
import jax.numpy as jnp
from jax import lax
from jax.experimental import pallas as pl
from jax.experimental.pallas import tpu as pltpu
from jax.experimental.pallas import tpu_sc as plsc

_D = 512
_NG = 50000
_NM = 10000
_NE = 150000
_RB = 1000

_NC = 2
_NS = 16
_NW = _NC * _NS

_G_CH = 48
_G_NCHUNK = _NE // _G_CH
_G_ITERS = (_G_NCHUNK + _NW - 1) // _NW

_S_CH = 120
_S_NCHUNK = _NE // _S_CH
_S_ITERS = (_S_NCHUNK + _NS - 1) // _NS
_A_RB = 632
_A_RB_LAST = _NM - 15 * _A_RB
_CW = 128


def _ln(y, scale, bias):
    mu = jnp.mean(y, axis=-1, keepdims=True)
    var = jnp.mean((y - mu) ** 2, axis=-1, keepdims=True)
    return (y - mu) * lax.rsqrt(var + 1e-5) * scale + bias


def _mlp(x, w1, b1, w2, b2, scale, bias):
    h = jnp.dot(x, w1, preferred_element_type=jnp.float32) + b1
    h = h * jax.nn.sigmoid(h)
    y = jnp.dot(h, w2, preferred_element_type=jnp.float32) + b2
    return _ln(y, scale, bias)


def _mesh_body(xm_ref, xg_ref,
               mW1, mb1, mW2, mb2, ms, mbb,
               gW1, gb1, gW2, gb2, gs, gbb,
               W1m, b1e, W1g,
               m_ref, pm_ref, pg_ref):
    m = _mlp(xm_ref[...], mW1[...], mb1[...], mW2[...], mb2[...], ms[...], mbb[...])
    m_ref[...] = m
    pm_ref[...] = jnp.dot(m, W1m[...], preferred_element_type=jnp.float32) + b1e[...]
    gh = _mlp(xg_ref[...], gW1[...], gb1[...], gW2[...], gb2[...], gs[...], gbb[...])
    pg_ref[...] = jnp.dot(gh, W1g[...], preferred_element_type=jnp.float32)


def _grid_body(x_ref,
               eW1, eb1, eW2, eb2, es, ebb,
               nW1, nb1, nW2, nb2, ns, nbb,
               out_ref):
    g = _mlp(x_ref[...], eW1[...], eb1[...], eW2[...], eb2[...], es[...], ebb[...])
    out_ref[...] = _mlp(g, nW1[...], nb1[...], nW2[...], nb2[...], ns[...], nbb[...]) + g


def _edge_body(ef_ref, sg_ref, sm_ref,
               eW1, eb1, eW2, eb2, es, ebb,
               W1e, fW2, fb2, fs, fbb,
               out_ref):
    e = _mlp(ef_ref[...], eW1[...], eb1[...], eW2[...], eb2[...], es[...], ebb[...])
    h = jnp.dot(e, W1e[...], preferred_element_type=jnp.float32) + sg_ref[...] + sm_ref[...]
    h = h * jax.nn.sigmoid(h)
    y = jnp.dot(h, fW2[...], preferred_element_type=jnp.float32) + fb2[...]
    out_ref[...] = _ln(y, fs[...], fbb[...]) + e


def _mesh2_body(m_ref, agg_ref,
                W1m2, W1a, b1m, mW2, mb2, ms2, mbb2,
                out_ref):
    m = m_ref[...]
    h = (jnp.dot(m, W1m2[...], preferred_element_type=jnp.float32)
         + jnp.dot(agg_ref[...], W1a[...], preferred_element_type=jnp.float32)
         + b1m[...])
    h = h * jax.nn.sigmoid(h)
    y = jnp.dot(h, mW2[...], preferred_element_type=jnp.float32) + mb2[...]
    out_ref[...] = _ln(y, ms2[...], mbb2[...]) + m


def _full(a):
    nd = a.ndim
    return pl.BlockSpec(a.shape, lambda i, _nd=nd: (0,) * _nd)


def _rows(width):
    return pl.BlockSpec((_RB, width), lambda i: (i, 0))


def _sc_gather(pg_tab, pm_tab, src, dst):
    mesh = plsc.VectorSubcoreMesh(core_axis_name="c", subcore_axis_name="s",
                                  num_cores=_NC, num_subcores=_NS)

    @functools.partial(
        pl.kernel,
        out_type=(jax.ShapeDtypeStruct((_NE, _D), jnp.float32),
                  jax.ShapeDtypeStruct((_NE, _D), jnp.float32)),
        mesh=mesh,
        scratch_types=[
            pltpu.VMEM((_G_CH,), jnp.int32),
            pltpu.VMEM((_G_CH,), jnp.int32),
            pltpu.VMEM((_G_CH, _D), jnp.float32),
            pltpu.VMEM((_G_CH, _D), jnp.float32),
            pltpu.SemaphoreType.DMA,
            pltpu.SemaphoreType.DMA,
        ],
    )
    def k(pg_hbm, pm_hbm, src_hbm, dst_hbm, sg_hbm, sm_hbm, ia, ib, ba, bb, sa, sb):
        w = lax.axis_index("s") * _NC + lax.axis_index("c")

        def body(i, carry):
            cid = w + i * _NW

            @pl.when(cid < _G_NCHUNK)
            def _():
                off = cid * _G_CH
                pltpu.sync_copy(src_hbm.at[pl.ds(off, _G_CH)], ia)
                pltpu.sync_copy(dst_hbm.at[pl.ds(off, _G_CH)], ib)
                ca = pltpu.async_copy(pg_hbm.at[ia], ba, sa)
                cb = pltpu.async_copy(pm_hbm.at[ib], bb, sb)
                ca.wait()
                cb.wait()
                pltpu.sync_copy(ba, sg_hbm.at[pl.ds(off, _G_CH)])
                pltpu.sync_copy(bb, sm_hbm.at[pl.ds(off, _G_CH)])

            return carry

        lax.fori_loop(0, _G_ITERS, body, 0)

    return k(pg_tab, pm_tab, src, dst)


def _sc_scatter(e_new, dst, zrows):
    mesh = plsc.VectorSubcoreMesh(core_axis_name="c", subcore_axis_name="s",
                                  num_cores=_NC, num_subcores=_NS)

    @functools.partial(
        pl.kernel,
        out_type=jax.ShapeDtypeStruct((_NM, _D), jnp.float32),
        mesh=mesh,
        scratch_types=[
            pltpu.VMEM((_S_CH,), jnp.int32),
            pltpu.VMEM((_S_CH, _CW), jnp.float32),
            pltpu.VMEM_SHARED((_NM, _CW), jnp.float32),
        ],
    )
    def k(enew_hbm, dst_hbm, z_hbm, agg_hbm, idx, buf, slab):
        c = lax.axis_index("c")
        s = lax.axis_index("s")
        for p in range(2):
            col0 = (c * 2 + p) * _CW

            @pl.when(s < 15)
            def _():
                pltpu.sync_copy(z_hbm, slab.at[pl.ds(s * _A_RB, _A_RB)])

            @pl.when(s == 15)
            def _():
                pltpu.sync_copy(z_hbm.at[pl.ds(0, _A_RB_LAST)],
                                slab.at[pl.ds(15 * _A_RB, _A_RB_LAST)])

            plsc.subcore_barrier()

            def body(i, carry):
                cid = s + i * _NS

                @pl.when(cid < _S_NCHUNK)
                def _():
                    off = cid * _S_CH
                    pltpu.sync_copy(dst_hbm.at[pl.ds(off, _S_CH)], idx)
                    pltpu.sync_copy(enew_hbm.at[pl.ds(off, _S_CH), pl.ds(col0, _CW)], buf)
                    pltpu.sync_copy(buf, slab.at[idx], add=True)

                return carry

            lax.fori_loop(0, _S_ITERS, body, 0)
            plsc.subcore_barrier()

            @pl.when(s < 15)
            def _():
                pltpu.sync_copy(slab.at[pl.ds(s * _A_RB, _A_RB)],
                                agg_hbm.at[pl.ds(s * _A_RB, _A_RB), pl.ds(col0, _CW)])

            @pl.when(s == 15)
            def _():
                pltpu.sync_copy(slab.at[pl.ds(15 * _A_RB, _A_RB_LAST)],
                                agg_hbm.at[pl.ds(15 * _A_RB, _A_RB_LAST), pl.ds(col0, _CW)])

            if p == 0:
                plsc.subcore_barrier()

    return k(e_new, dst, zrows)


def kernel(grid_nfeat, mesh_nfeat, edge_index, grid2mesh_efeat, params):
    src = edge_index[0].astype(jnp.int32)
    dst = edge_index[1].astype(jnp.int32)

    def v2(a):
        return a.reshape(1, -1)

    eg, em, ee = params["emb_grid"], params["emb_mesh"], params["emb_edge"]
    pe, pn, pg = params["edge_mlp"], params["mesh_node_mlp"], params["grid_node_mlp"]
    W1g, W1m, W1e = pe["W1"][:_D], pe["W1"][_D:2 * _D], pe["W1"][2 * _D:]
    W1m2, W1a = pn["W1"][:_D], pn["W1"][_D:]

    def wpack(p):
        return (p["W1"], v2(p["b1"]), p["W2"], v2(p["b2"]),
                v2(p["ln_scale"]), v2(p["ln_bias"]))

    mesh_in = (mesh_nfeat, grid_nfeat[:_NM]) + wpack(em) + wpack(eg) \
        + (W1m, v2(pe["b1"]), W1g)
    m, pm_tab, pg_tab = pl.pallas_call(
        _mesh_body,
        grid=(_NM // _RB,),
        in_specs=[_rows(3), _rows(474)] + [_full(a) for a in mesh_in[2:]],
        out_specs=(_rows(_D), _rows(_D), _rows(_D)),
        out_shape=(jax.ShapeDtypeStruct((_NM, _D), jnp.float32),
                   jax.ShapeDtypeStruct((_NM, _D), jnp.float32),
                   jax.ShapeDtypeStruct((_NM, _D), jnp.float32)),
    )(*mesh_in)

    grid_in = (grid_nfeat,) + wpack(eg) + wpack(pg)
    g_new = pl.pallas_call(
        _grid_body,
        grid=(_NG // _RB,),
        in_specs=[_rows(474)] + [_full(a) for a in grid_in[1:]],
        out_specs=_rows(_D),
        out_shape=jax.ShapeDtypeStruct((_NG, _D), jnp.float32),
    )(*grid_in)

    s_g, s_m = _sc_gather(pg_tab, pm_tab, src, dst)

    edge_in = (grid2mesh_efeat, s_g, s_m) + wpack(ee) \
        + (W1e, pe["W2"], v2(pe["b2"]), v2(pe["ln_scale"]), v2(pe["ln_bias"]))
    e_new = pl.pallas_call(
        _edge_body,
        grid=(_NE // _RB,),
        in_specs=[_rows(4), _rows(_D), _rows(_D)] + [_full(a) for a in edge_in[3:]],
        out_specs=_rows(_D),
        out_shape=jax.ShapeDtypeStruct((_NE, _D), jnp.float32),
    )(*edge_in)

    zrows = jnp.zeros((_A_RB, _CW), jnp.float32)
    agg = _sc_scatter(e_new, dst, zrows)

    mesh2_in = (m, agg, W1m2, W1a, v2(pn["b1"]), pn["W2"], v2(pn["b2"]),
                v2(pn["ln_scale"]), v2(pn["ln_bias"]))
    m_new = pl.pallas_call(
        _mesh2_body,
        grid=(_NM // _RB,),
        in_specs=[_rows(_D), _rows(_D)] + [_full(a) for a in mesh2_in[2:]],
        out_specs=_rows(_D),
        out_shape=jax.ShapeDtypeStruct((_NM, _D), jnp.float32),
    )(*mesh2_in)

    return g_new, m_new, e_new


# speedup vs baseline: 3.0767x; 1.0114x over previous
import functools

import jax
import jax.numpy as jnp
from jax import lax
from jax.experimental import pallas as pl
from jax.experimental.pallas import tpu as pltpu
from jax.experimental.pallas import tpu_sc as plsc

_D = 512
_NG = 50000
_NM = 10000
_NE = 150000
_RB = 1000
_RB_M = 2000
_RB_E = 1200

_NC = 2
_NS = 16
_NW = _NC * _NS

_G_CH = 80
_G_NCHUNK = _NE // _G_CH
_G_PAIRS = (_G_NCHUNK + 2 * _NW - 1) // (2 * _NW)

_S_CH = 120
_S_NCHUNK = _NE // _S_CH
_S_PAIRS = (_S_NCHUNK + 2 * _NS - 1) // (2 * _NS)
_A_RB = 632
_A_RB_LAST = _NM - 15 * _A_RB
_CW = 128


def _ln(y, scale, bias):
    mu = jnp.mean(y, axis=-1, keepdims=True)
    var = jnp.mean((y - mu) ** 2, axis=-1, keepdims=True)
    return (y - mu) * lax.rsqrt(var + 1e-5) * scale + bias


def _mlp(x, w1, b1, w2, b2, scale, bias):
    h = jnp.dot(x, w1, preferred_element_type=jnp.float32) + b1
    h = h * jax.nn.sigmoid(h)
    y = jnp.dot(h, w2, preferred_element_type=jnp.float32) + b2
    return _ln(y, scale, bias)


def _mesh_body(xm_ref, xg_ref,
               mW1, mb1, mW2, mb2, ms, mbb,
               gW1, gb1, gW2, gb2, gs, gbb,
               W1m, b1e, W1g,
               m_ref, pm_ref, pg_ref):
    m = _mlp(xm_ref[...], mW1[...], mb1[...], mW2[...], mb2[...], ms[...], mbb[...])
    m_ref[...] = m
    pm = jnp.dot(m, W1m[...], preferred_element_type=jnp.float32) + b1e[...]
    pm_ref[...] = pm.astype(jnp.bfloat16)
    gh = _mlp(xg_ref[...], gW1[...], gb1[...], gW2[...], gb2[...], gs[...], gbb[...])
    pg_ref[...] = jnp.dot(gh, W1g[...], preferred_element_type=jnp.float32).astype(jnp.bfloat16)


def _grid_body(x_ref,
               eW1, eb1, eW2, eb2, es, ebb,
               nW1, nb1, nW2, nb2, ns, nbb,
               out_ref):
    g = _mlp(x_ref[...], eW1[...], eb1[...], eW2[...], eb2[...], es[...], ebb[...])
    out_ref[...] = _mlp(g, nW1[...], nb1[...], nW2[...], nb2[...], ns[...], nbb[...]) + g


def _unpack2(v):
    lo = lax.bitcast_convert_type(v << 16, jnp.float32)
    hi = lax.bitcast_convert_type((v >> 16) << 16, jnp.float32)
    return lo, hi


def _edge_body(ef_ref, sg_ref, sm_ref,
               eW1, eb1, eW2, eb2, es, ebb,
               W1e, fW2, fb2, fs, fbb,
               out_ref):
    e = _mlp(ef_ref[...], eW1[...], eb1[...], eW2[...], eb2[...], es[...], ebb[...])
    g_lo, g_hi = _unpack2(sg_ref[...])
    m_lo, m_hi = _unpack2(sm_ref[...])
    s = jnp.concatenate([g_lo + m_lo, g_hi + m_hi], axis=-1)
    h = jnp.dot(e, W1e[...], preferred_element_type=jnp.float32) + s
    h = h * jax.nn.sigmoid(h)
    y = jnp.dot(h, fW2[...], preferred_element_type=jnp.float32) + fb2[...]
    out_ref[...] = _ln(y, fs[...], fbb[...]) + e


def _mesh2_body(m_ref, agg_ref,
                W1m2, W1a, b1m, mW2, mb2, ms2, mbb2,
                out_ref):
    m = m_ref[...]
    h = (jnp.dot(m, W1m2[...], preferred_element_type=jnp.float32)
         + jnp.dot(agg_ref[...], W1a[...], preferred_element_type=jnp.float32)
         + b1m[...])
    h = h * jax.nn.sigmoid(h)
    y = jnp.dot(h, mW2[...], preferred_element_type=jnp.float32) + mb2[...]
    out_ref[...] = _ln(y, ms2[...], mbb2[...]) + m


def _full(a):
    nd = a.ndim
    return pl.BlockSpec(a.shape, lambda i, _nd=nd: (0,) * _nd)


def _rows(width, rb=_RB):
    return pl.BlockSpec((rb, width), lambda i: (i, 0))


def _sc_gather(pg_tab, pm_tab, src, dst):
    mesh = plsc.VectorSubcoreMesh(core_axis_name="c", subcore_axis_name="s",
                                  num_cores=_NC, num_subcores=_NS)

    @functools.partial(
        pl.kernel,
        out_type=(jax.ShapeDtypeStruct((_NE, _D // 2), jnp.int32),
                  jax.ShapeDtypeStruct((_NE, _D // 2), jnp.int32)),
        mesh=mesh,
        scratch_types=[
            pltpu.VMEM((_G_CH,), jnp.int32),
            pltpu.VMEM((_G_CH,), jnp.int32),
            pltpu.VMEM((_G_CH,), jnp.int32),
            pltpu.VMEM((_G_CH,), jnp.int32),
            pltpu.VMEM((_G_CH, _D // 2), jnp.int32),
            pltpu.VMEM((_G_CH, _D // 2), jnp.int32),
            pltpu.VMEM((_G_CH, _D // 2), jnp.int32),
            pltpu.VMEM((_G_CH, _D // 2), jnp.int32),
            pltpu.SemaphoreType.DMA,
            pltpu.SemaphoreType.DMA,
            pltpu.SemaphoreType.DMA,
            pltpu.SemaphoreType.DMA,
        ],
    )
    def k(pg_hbm, pm_hbm, src_hbm, dst_hbm, sg_hbm, sm_hbm,
          ia0, ib0, ia1, ib1, ba0, bb0, ba1, bb1, sa0, sb0, sa1, sb1):
        w = lax.axis_index("s") * _NC + lax.axis_index("c")

        def fire(cid, ia, ib, ba, bb, sa, sb):
            off = cid * _G_CH
            pltpu.sync_copy(src_hbm.at[pl.ds(off, _G_CH)], ia)
            pltpu.sync_copy(dst_hbm.at[pl.ds(off, _G_CH)], ib)
            pltpu.async_copy(pg_hbm.at[ia], ba, sa)
            pltpu.async_copy(pm_hbm.at[ib], bb, sb)

        def drain(cid, ia, ib, ba, bb, sa, sb):
            off = cid * _G_CH
            pltpu.make_async_copy(pg_hbm.at[ia], ba, sa).wait()
            pltpu.make_async_copy(pm_hbm.at[ib], bb, sb).wait()
            pltpu.sync_copy(ba, sg_hbm.at[pl.ds(off, _G_CH)])
            pltpu.sync_copy(bb, sm_hbm.at[pl.ds(off, _G_CH)])

        def body(j, carry):
            c0 = w + (2 * j) * _NW
            c1 = w + (2 * j + 1) * _NW

            @pl.when(c0 < _G_NCHUNK)
            def _():
                fire(c0, ia0, ib0, ba0, bb0, sa0, sb0)

            @pl.when(c1 < _G_NCHUNK)
            def _():
                fire(c1, ia1, ib1, ba1, bb1, sa1, sb1)

            @pl.when(c0 < _G_NCHUNK)
            def _():
                drain(c0, ia0, ib0, ba0, bb0, sa0, sb0)

            @pl.when(c1 < _G_NCHUNK)
            def _():
                drain(c1, ia1, ib1, ba1, bb1, sa1, sb1)

            return carry

        lax.fori_loop(0, _G_PAIRS, body, 0)

    return k(pg_tab, pm_tab, src, dst)


def _sc_scatter(e_new, dst, zrows):
    mesh = plsc.VectorSubcoreMesh(core_axis_name="c", subcore_axis_name="s",
                                  num_cores=_NC, num_subcores=_NS)

    @functools.partial(
        pl.kernel,
        out_type=jax.ShapeDtypeStruct((_NM, _D), jnp.float32),
        mesh=mesh,
        scratch_types=[
            pltpu.VMEM((_S_CH,), jnp.int32),
            pltpu.VMEM((_S_CH,), jnp.int32),
            pltpu.VMEM((_S_CH, _CW), jnp.float32),
            pltpu.VMEM((_S_CH, _CW), jnp.float32),
            pltpu.VMEM_SHARED((_NM, _CW), jnp.float32),
            pltpu.SemaphoreType.DMA,
            pltpu.SemaphoreType.DMA,
        ],
    )
    def k(enew_hbm, dst_hbm, z_hbm, agg_hbm, idx0, idx1, buf0, buf1, slab, se0, se1):
        c = lax.axis_index("c")
        s = lax.axis_index("s")
        for p in range(2):
            col0 = (c * 2 + p) * _CW

            @pl.when(s < 15)
            def _():
                pltpu.sync_copy(z_hbm, slab.at[pl.ds(s * _A_RB, _A_RB)])

            @pl.when(s == 15)
            def _():
                pltpu.sync_copy(z_hbm.at[pl.ds(0, _A_RB_LAST)],
                                slab.at[pl.ds(15 * _A_RB, _A_RB_LAST)])

            plsc.subcore_barrier()

            def fire(cid, idx, buf, sem):
                off = cid * _S_CH
                pltpu.sync_copy(dst_hbm.at[pl.ds(off, _S_CH)], idx)
                pltpu.async_copy(enew_hbm.at[pl.ds(off, _S_CH), pl.ds(col0, _CW)],
                                 buf, sem)

            def drain(cid, idx, buf, sem):
                off = cid * _S_CH
                pltpu.make_async_copy(
                    enew_hbm.at[pl.ds(off, _S_CH), pl.ds(col0, _CW)], buf, sem).wait()
                pltpu.sync_copy(buf, slab.at[idx], add=True)

            def body(j, carry):
                c0 = s + (2 * j) * _NS
                c1 = s + (2 * j + 1) * _NS

                @pl.when(c0 < _S_NCHUNK)
                def _():
                    fire(c0, idx0, buf0, se0)

                @pl.when(c1 < _S_NCHUNK)
                def _():
                    fire(c1, idx1, buf1, se1)

                @pl.when(c0 < _S_NCHUNK)
                def _():
                    drain(c0, idx0, buf0, se0)

                @pl.when(c1 < _S_NCHUNK)
                def _():
                    drain(c1, idx1, buf1, se1)

                return carry

            lax.fori_loop(0, _S_PAIRS, body, 0)
            plsc.subcore_barrier()

            @pl.when(s < 15)
            def _():
                pltpu.sync_copy(slab.at[pl.ds(s * _A_RB, _A_RB)],
                                agg_hbm.at[pl.ds(s * _A_RB, _A_RB), pl.ds(col0, _CW)])

            @pl.when(s == 15)
            def _():
                pltpu.sync_copy(slab.at[pl.ds(15 * _A_RB, _A_RB_LAST)],
                                agg_hbm.at[pl.ds(15 * _A_RB, _A_RB_LAST), pl.ds(col0, _CW)])

            if p == 0:
                plsc.subcore_barrier()

    return k(e_new, dst, zrows)


def kernel(grid_nfeat, mesh_nfeat, edge_index, grid2mesh_efeat, params):
    src = edge_index[0].astype(jnp.int32)
    dst = edge_index[1].astype(jnp.int32)

    def v2(a):
        return a.reshape(1, -1)

    eg, em, ee = params["emb_grid"], params["emb_mesh"], params["emb_edge"]
    pe, pn, pg = params["edge_mlp"], params["mesh_node_mlp"], params["grid_node_mlp"]
    W1g, W1m, W1e = pe["W1"][:_D], pe["W1"][_D:2 * _D], pe["W1"][2 * _D:]
    W1m2, W1a = pn["W1"][:_D], pn["W1"][_D:]

    def wpack(p):
        return (p["W1"], v2(p["b1"]), p["W2"], v2(p["b2"]),
                v2(p["ln_scale"]), v2(p["ln_bias"]))

    mesh_in = (mesh_nfeat, grid_nfeat[:_NM]) + wpack(em) + wpack(eg) \
        + (W1m, v2(pe["b1"]), W1g)
    m, pm_tab, pg_tab = pl.pallas_call(
        _mesh_body,
        grid=(_NM // _RB_M,),
        in_specs=[_rows(3, _RB_M), _rows(474, _RB_M)] + [_full(a) for a in mesh_in[2:]],
        out_specs=(_rows(_D, _RB_M), _rows(_D, _RB_M), _rows(_D, _RB_M)),
        out_shape=(jax.ShapeDtypeStruct((_NM, _D), jnp.float32),
                   jax.ShapeDtypeStruct((_NM, _D), jnp.bfloat16),
                   jax.ShapeDtypeStruct((_NM, _D), jnp.bfloat16)),
    )(*mesh_in)

    grid_in = (grid_nfeat,) + wpack(eg) + wpack(pg)
    g_new = pl.pallas_call(
        _grid_body,
        grid=(_NG // _RB,),
        in_specs=[_rows(474)] + [_full(a) for a in grid_in[1:]],
        out_specs=_rows(_D),
        out_shape=jax.ShapeDtypeStruct((_NG, _D), jnp.float32),
    )(*grid_in)

    def pack2(t):
        return lax.bitcast_convert_type(t.reshape(_NM, _D // 2, 2), jnp.int32)

    s_g, s_m = _sc_gather(pack2(pg_tab), pack2(pm_tab), src, dst)

    tau = jnp.concatenate([jnp.arange(0, _D, 2), jnp.arange(1, _D, 2)])

    edge_in = (grid2mesh_efeat, s_g, s_m) + wpack(ee) \
        + (W1e[:, tau], pe["W2"][tau, :], v2(pe["b2"]),
           v2(pe["ln_scale"]), v2(pe["ln_bias"]))
    e_new = pl.pallas_call(
        _edge_body,
        grid=(_NE // _RB_E,),
        in_specs=[_rows(4, _RB_E), _rows(_D // 2, _RB_E), _rows(_D // 2, _RB_E)]
        + [_full(a) for a in edge_in[3:]],
        out_specs=_rows(_D, _RB_E),
        out_shape=jax.ShapeDtypeStruct((_NE, _D), jnp.float32),
    )(*edge_in)

    zrows = jnp.zeros((_A_RB, _CW), jnp.float32)
    agg = _sc_scatter(e_new, dst, zrows)

    mesh2_in = (m, agg, W1m2, W1a, v2(pn["b1"]), pn["W2"], v2(pn["b2"]),
                v2(pn["ln_scale"]), v2(pn["ln_bias"]))
    m_new = pl.pallas_call(
        _mesh2_body,
        grid=(_NM // _RB,),
        in_specs=[_rows(_D), _rows(_D)] + [_full(a) for a in mesh2_in[2:]],
        out_specs=_rows(_D),
        out_shape=jax.ShapeDtypeStruct((_NM, _D), jnp.float32),
    )(*mesh2_in)

    return g_new, m_new, e_new
